# baseline (speedup 1.0000x reference)
import functools

import jax
import jax.numpy as jnp
from jax import lax
from jax.experimental import pallas as pl
from jax.experimental.pallas import tpu as pltpu
from jax.experimental.pallas import tpu_sc as plsc

D_MODEL = 1024
BATCH = 4
SEQ = 2048
B = BATCH * SEQ

_NC = 2
_NS = 16
_NW = _NC * _NS
_BPW = B // _NW
_CHUNKS = (64, 56, 64, 56, 8, 8)
_BUFROWS = 120

_mesh = plsc.VectorSubcoreMesh(core_axis_name="c", subcore_axis_name="s")


@functools.partial(
    pl.kernel,
    mesh=_mesh,
    out_type=jax.ShapeDtypeStruct((B, D_MODEL), jnp.float32),
    scratch_types=[
        pltpu.VMEM((_BPW,), jnp.int32),
        pltpu.VMEM((_BUFROWS, D_MODEL), jnp.float32),
        pltpu.SemaphoreType.DMA,
        pltpu.SemaphoreType.DMA,
        pltpu.SemaphoreType.DMA,
    ],
)
def _embed_sc(ids_hbm, table_hbm, out_hbm, idx_v, rows_v, gsem, osem0, osem1):
    wid = lax.axis_index("s") * _NC + lax.axis_index("c")
    base = wid * _BPW

    offs = []
    o = 0
    for c in _CHUNKS:
        offs.append(o)
        o += c

    def buf(i):
        return rows_v.at[pl.ds(64 * (i % 2), _CHUNKS[i])]

    def gather(i):
        return pltpu.async_copy(
            table_hbm.at[idx_v.at[pl.ds(offs[i], _CHUNKS[i])]], buf(i), gsem
        )

    def put(i):
        return pltpu.async_copy(
            buf(i),
            out_hbm.at[pl.ds(base + offs[i], _CHUNKS[i])],
            osem0 if i % 2 == 0 else osem1,
        )

    n = len(_CHUNKS)
    pltpu.sync_copy(ids_hbm.at[pl.ds(base, _BPW)], idx_v)
    gathers = [gather(0)]
    puts = []
    for i in range(n):
        gathers[i].wait()
        puts.append(put(i))
        if i + 1 < n:
            if i >= 1:
                puts[i - 1].wait()
            gathers.append(gather(i + 1))
    puts[n - 2].wait()
    puts[n - 1].wait()


def kernel(text_ids, table):
    ids_flat = text_ids.reshape(-1).astype(jnp.int32)
    out = _embed_sc(ids_flat, table)
    return out.reshape(BATCH, SEQ, D_MODEL)
